# baseline (speedup 1.0000x reference)
import functools

import jax
import jax.numpy as jnp
from jax import lax
from jax.experimental import pallas as pl
from jax.experimental.pallas import tpu as pltpu
from jax.experimental.pallas import tpu_sc as plsc

N = 10000
E = 320000
D_IN = 128
D_H = 128
D_OUT = 64

NC = 2
NS = 16

N_ACC = 10240
E_PAD = 327680
EROWS = E_PAD // 128
ROWS_PER_TILE = EROWS // (NC * NS)


SEG_CHUNK = 128
STEPS = E_PAD // (NC * NS * SEG_CHUNK)


def _make_seg_sum(d):
    mesh = plsc.VectorSubcoreMesh(core_axis_name="c", subcore_axis_name="s")

    @functools.partial(
        pl.kernel,
        out_type=jax.ShapeDtypeStruct((NC, N, d), jnp.float32),
        mesh=mesh,
        compiler_params=pltpu.CompilerParams(use_tc_tiling_on_sc=False),
        scratch_types=[
            pltpu.VMEM_SHARED((N_ACC, d), jnp.float32),
            pltpu.VMEM((2, SEG_CHUNK), jnp.int32),
            pltpu.VMEM((2, SEG_CHUNK), jnp.int32),
            pltpu.VMEM((2, SEG_CHUNK), jnp.int32),
            pltpu.VMEM((2, SEG_CHUNK), jnp.int32),
            pltpu.VMEM((SEG_CHUNK, d), jnp.float32),
            pltpu.VMEM((SEG_CHUNK, d), jnp.float32),
            pltpu.SemaphoreType.DMA,
            pltpu.SemaphoreType.DMA,
            pltpu.SemaphoreType.DMA,
        ],
    )
    def seg_sum(table, e2, zeros, out, acc, i0, i1, i2, i3, r0, r1,
                isem, gsem, asem):
        c = lax.axis_index("c")
        s = lax.axis_index("s")
        idxb = [i0, i1, i2, i3]
        rows = [r0, r1]
        zr = N_ACC // NS
        pltpu.sync_copy(zeros.at[pl.ds(s * zr, zr)], acc.at[pl.ds(s * zr, zr)])
        plsc.subcore_barrier()
        t0 = (c * NS + s) * STEPS

        def fire_idx(e, et):
            pltpu.async_copy(e2.at[t0 + et], idxb[e % 4], isem)

        def wait_idx(e):
            pltpu.make_async_copy(e2.at[t0], idxb[e % 4], isem).wait()

        def fire_gather(e):
            pltpu.async_copy(table.at[idxb[e % 4].at[0]], rows[e % 2], gsem)

        def wait_gather(e):
            pltpu.make_async_copy(table.at[idxb[e % 4].at[0]], rows[e % 2],
                                  gsem).wait()

        def fire_add(e):
            pltpu.async_copy(rows[e % 2], acc.at[idxb[e % 4].at[1]], asem,
                             add=True)

        def wait_add(e):
            pltpu.make_async_copy(rows[e % 2], acc.at[idxb[e % 4].at[1]],
                                  asem).wait()

        def st(e, et, *, wait_prev=True, next_gather=True, next2_idx=True):
            wait_gather(e)
            fire_add(e)
            if wait_prev:
                wait_add(e - 1)
            if next_gather:
                wait_idx(e + 1)
                fire_gather(e + 1)
            if next2_idx:
                fire_idx(e + 2, et + 2)

        pltpu.sync_copy(e2.at[t0], i0)
        fire_gather(0)
        fire_idx(1, 1)
        st(0, 0, wait_prev=False)
        st(1, 1)
        st(2, 2)
        st(3, 3)

        def body(g, carry):
            et = 4 * g
            st(0, et)
            st(1, et + 1)
            st(2, et + 2)
            st(3, et + 3)
            return carry

        lax.fori_loop(1, STEPS // 4 - 1, body, 0)

        st(0, STEPS - 4)
        st(1, STEPS - 3)
        st(2, STEPS - 2, next2_idx=False)
        st(3, STEPS - 1, next_gather=False, next2_idx=False)
        wait_add(3)

        plsc.subcore_barrier()
        @pl.when(s < 15)
        def _():
            pltpu.sync_copy(acc.at[pl.ds(s * 624, 624)],
                            out.at[c, pl.ds(s * 624, 624)])

        @pl.when(s == 15)
        def _():
            pltpu.sync_copy(acc.at[pl.ds(9360, 640)],
                            out.at[c, pl.ds(9360, 640)])

    return seg_sum


def _make_deg():
    mesh = plsc.VectorSubcoreMesh(core_axis_name="c", subcore_axis_name="s")

    @functools.partial(
        pl.kernel,
        out_type=jax.ShapeDtypeStruct((NC, N, 16), jnp.float32),
        mesh=mesh,
        compiler_params=pltpu.CompilerParams(use_tc_tiling_on_sc=False),
        scratch_types=[
            pltpu.VMEM_SHARED((N_ACC, 16), jnp.float32),
            pltpu.VMEM((8, 128), jnp.int32),
            pltpu.VMEM((128, 16), jnp.float32),
            pltpu.SemaphoreType.DMA,
        ],
    )
    def deg_kernel(dst2d, ones_hbm, zeros16, out, accd, idx_d, ones_v, asem):
        c = lax.axis_index("c")
        s = lax.axis_index("s")
        zr = N_ACC // NS
        pltpu.sync_copy(zeros16.at[pl.ds(s * zr, zr)],
                        accd.at[pl.ds(s * zr, zr)])
        pltpu.sync_copy(ones_hbm, ones_v)
        plsc.subcore_barrier()
        tile_row0 = (c * NS + s) * ROWS_PER_TILE

        def body(g, carry):
            pltpu.sync_copy(dst2d.at[pl.ds(tile_row0 + g * 8, 8)], idx_d)
            descs = [
                pltpu.async_copy(ones_v, accd.at[idx_d.at[j]], asem,
                                 add=True)
                for j in range(8)
            ]
            for dsc in descs:
                dsc.wait()
            return carry

        lax.fori_loop(0, ROWS_PER_TILE // 8, body, 0)
        plsc.subcore_barrier()

        @pl.when(s < 15)
        def _():
            pltpu.sync_copy(accd.at[pl.ds(s * 624, 624)],
                            out.at[c, pl.ds(s * 624, 624)])

        @pl.when(s == 15)
        def _():
            pltpu.sync_copy(accd.at[pl.ds(9360, 640)],
                            out.at[c, pl.ds(9360, 640)])

    return deg_kernel


_seg128 = _make_seg_sum(128)
_seg64 = _make_seg_sum(64)
_deg = _make_deg()

_TC_R = 2000


def _tc_first(x, ws, wn, b):
    def body(x_ref, ws_ref, wn_ref, b_ref, self_ref, xn_ref):
        xb = x_ref[...]
        self_ref[...] = jnp.dot(
            xb, ws_ref[...], preferred_element_type=jnp.float32) + b_ref[...]
        xn_ref[...] = jnp.dot(
            xb, wn_ref[...], preferred_element_type=jnp.float32)

    dk, do = ws.shape
    return pl.pallas_call(
        body,
        grid=(N // _TC_R,),
        in_specs=[
            pl.BlockSpec((_TC_R, dk), lambda i: (i, 0)),
            pl.BlockSpec((dk, do), lambda i: (0, 0)),
            pl.BlockSpec((dk, do), lambda i: (0, 0)),
            pl.BlockSpec((1, do), lambda i: (0, 0)),
        ],
        out_specs=[
            pl.BlockSpec((_TC_R, do), lambda i: (i, 0)),
            pl.BlockSpec((_TC_R, do), lambda i: (i, 0)),
        ],
        out_shape=[
            jax.ShapeDtypeStruct((N, do), jnp.float32),
            jax.ShapeDtypeStruct((N, do), jnp.float32),
        ],
    )(x, ws, wn, b)


def _tc_mid(selfp, sums, dega, degb, ws, wn, b):
    def body(self_ref, sums_ref, da_ref, db_ref, ws_ref, wn_ref, b_ref,
             sout_ref, xn_ref):
        deg = jnp.maximum(da_ref[...] + db_ref[...], 1.0)
        agg = (sums_ref[0] + sums_ref[1]) / deg
        h = jnp.maximum(self_ref[...] + agg, 0.0)
        sout_ref[...] = jnp.dot(
            h, ws_ref[...], preferred_element_type=jnp.float32) + b_ref[...]
        xn_ref[...] = jnp.dot(
            h, wn_ref[...], preferred_element_type=jnp.float32)

    dk, do = ws.shape
    return pl.pallas_call(
        body,
        grid=(N // _TC_R,),
        in_specs=[
            pl.BlockSpec((_TC_R, dk), lambda i: (i, 0)),
            pl.BlockSpec((NC, _TC_R, dk), lambda i: (0, i, 0)),
            pl.BlockSpec((_TC_R, 1), lambda i: (i, 0)),
            pl.BlockSpec((_TC_R, 1), lambda i: (i, 0)),
            pl.BlockSpec((dk, do), lambda i: (0, 0)),
            pl.BlockSpec((dk, do), lambda i: (0, 0)),
            pl.BlockSpec((1, do), lambda i: (0, 0)),
        ],
        out_specs=[
            pl.BlockSpec((_TC_R, do), lambda i: (i, 0)),
            pl.BlockSpec((_TC_R, do), lambda i: (i, 0)),
        ],
        out_shape=[
            jax.ShapeDtypeStruct((N, do), jnp.float32),
            jax.ShapeDtypeStruct((N, do), jnp.float32),
        ],
    )(selfp, sums, dega, degb, ws, wn, b)


def _tc_final(selfp, sums, dega, degb):
    def body(self_ref, sums_ref, da_ref, db_ref, out_ref):
        deg = jnp.maximum(da_ref[...] + db_ref[...], 1.0)
        out_ref[...] = self_ref[...] + (sums_ref[0] + sums_ref[1]) / deg

    do = selfp.shape[1]
    return pl.pallas_call(
        body,
        grid=(N // _TC_R,),
        in_specs=[
            pl.BlockSpec((_TC_R, do), lambda i: (i, 0)),
            pl.BlockSpec((NC, _TC_R, do), lambda i: (0, i, 0)),
            pl.BlockSpec((_TC_R, 1), lambda i: (i, 0)),
            pl.BlockSpec((_TC_R, 1), lambda i: (i, 0)),
        ],
        out_specs=pl.BlockSpec((_TC_R, do), lambda i: (i, 0)),
        out_shape=jax.ShapeDtypeStruct((N, do), jnp.float32),
    )(selfp, sums, dega, degb)


def kernel(in_feat, edge_index, W_self0, W_neigh0, b0, W_self1, W_neigh1,
           b1, W_self2, W_neigh2, b2):
    pad = E_PAD - E
    src_p = jnp.concatenate([edge_index[0], jnp.zeros((pad,), jnp.int32)])
    dst_p = jnp.concatenate([edge_index[1], jnp.full((pad,), N, jnp.int32)])
    dst2d = dst_p.reshape(EROWS, 128)
    e2 = jnp.stack([
        src_p.reshape(-1, SEG_CHUNK),
        dst_p.reshape(-1, SEG_CHUNK),
    ], axis=1)
    zeros128 = jnp.zeros((N_ACC, 128), jnp.float32)
    zeros64 = jnp.zeros((N_ACC, 64), jnp.float32)
    zeros16 = jnp.zeros((N_ACC, 16), jnp.float32)
    ones16 = jnp.ones((128, 16), jnp.float32)

    degs = _deg(dst2d, ones16, zeros16)
    dega = degs[0, :, 0:1]
    degb = degs[1, :, 0:1]

    self0, xn0 = _tc_first(in_feat, W_self0, W_neigh0, b0.reshape(1, -1))
    sums0 = _seg128(xn0, e2, zeros128)
    self1, xn1 = _tc_mid(self0, sums0, dega, degb, W_self1, W_neigh1,
                         b1.reshape(1, -1))
    sums1 = _seg128(xn1, e2, zeros128)
    self2, xn2 = _tc_mid(self1, sums1, dega, degb, W_self2, W_neigh2,
                         b2.reshape(1, -1))
    sums2 = _seg64(xn2, e2, zeros64)
    return _tc_final(self2, sums2, dega, degb)


# speedup vs baseline: 1.4640x; 1.4640x over previous
import functools

import jax
import jax.numpy as jnp
from jax import lax
from jax.experimental import pallas as pl
from jax.experimental.pallas import tpu as pltpu
from jax.experimental.pallas import tpu_sc as plsc

N = 10000
E = 320000
D_IN = 128
D_H = 128
D_OUT = 64

NC = 2
NS = 16

N_ACC = 10240
E_PAD = 327680
EROWS = E_PAD // 128
ROWS_PER_TILE = EROWS // (NC * NS)


SEG_CHUNK = 512
STEPS = E_PAD // (NS * SEG_CHUNK)


def _make_seg_sum(w):
    mesh = plsc.VectorSubcoreMesh(core_axis_name="c", subcore_axis_name="s")

    @functools.partial(
        pl.kernel,
        out_type=jax.ShapeDtypeStruct((NC, N, w), jnp.float32),
        mesh=mesh,
        compiler_params=pltpu.CompilerParams(use_tc_tiling_on_sc=False),
        scratch_types=[
            pltpu.VMEM_SHARED((N_ACC, w), jnp.float32),
            pltpu.VMEM((2, SEG_CHUNK), jnp.int32),
            pltpu.VMEM((2, SEG_CHUNK), jnp.int32),
            pltpu.VMEM((2, SEG_CHUNK), jnp.int32),
            pltpu.VMEM((2, SEG_CHUNK), jnp.int32),
            pltpu.VMEM((SEG_CHUNK, w), jnp.float32),
            pltpu.VMEM((SEG_CHUNK, w), jnp.float32),
            pltpu.SemaphoreType.DMA,
            pltpu.SemaphoreType.DMA,
            pltpu.SemaphoreType.DMA,
        ],
    )
    def seg_sum(tablea, tableb, e2, zeros, out, acc, i0, i1, i2, i3, r0, r1,
                isem, gsem, asem):
        c = lax.axis_index("c")
        s = lax.axis_index("s")
        idxb = [i0, i1, i2, i3]
        rows = [r0, r1]
        zr = N_ACC // NS
        pltpu.sync_copy(zeros.at[pl.ds(s * zr, zr)], acc.at[pl.ds(s * zr, zr)])
        plsc.subcore_barrier()
        t0 = s * STEPS

        def fire_idx(e, et):
            pltpu.async_copy(e2.at[t0 + et], idxb[e % 4], isem)

        def wait_idx(e):
            pltpu.make_async_copy(e2.at[t0], idxb[e % 4], isem).wait()

        def fire_gather(e):
            @pl.when(c == 0)
            def _():
                pltpu.async_copy(tablea.at[idxb[e % 4].at[0]], rows[e % 2],
                                 gsem)

            @pl.when(c == 1)
            def _():
                pltpu.async_copy(tableb.at[idxb[e % 4].at[0]], rows[e % 2],
                                 gsem)

        def wait_gather(e):
            pltpu.make_async_copy(tablea.at[idxb[e % 4].at[0]], rows[e % 2],
                                  gsem).wait()

        def fire_add(e):
            pltpu.async_copy(rows[e % 2], acc.at[idxb[e % 4].at[1]], asem,
                             add=True)

        def wait_add(e):
            pltpu.make_async_copy(rows[e % 2], acc.at[idxb[e % 4].at[1]],
                                  asem).wait()

        def st(e, et, *, wait_prev=True, next_gather=True, next2_idx=True):
            wait_gather(e)
            fire_add(e)
            if wait_prev:
                wait_add(e - 1)
            if next_gather:
                wait_idx(e + 1)
                fire_gather(e + 1)
            if next2_idx:
                fire_idx(e + 2, et + 2)

        pltpu.sync_copy(e2.at[t0], i0)
        fire_gather(0)
        fire_idx(1, 1)
        st(0, 0, wait_prev=False)
        st(1, 1)
        st(2, 2)
        st(3, 3)

        def body(g, carry):
            et = 4 * g
            st(0, et)
            st(1, et + 1)
            st(2, et + 2)
            st(3, et + 3)
            return carry

        lax.fori_loop(1, STEPS // 4 - 1, body, 0)

        st(0, STEPS - 4)
        st(1, STEPS - 3)
        st(2, STEPS - 2, next2_idx=False)
        st(3, STEPS - 1, next_gather=False, next2_idx=False)
        wait_add(3)

        plsc.subcore_barrier()
        @pl.when(s < 15)
        def _():
            pltpu.sync_copy(acc.at[pl.ds(s * 624, 624)],
                            out.at[c, pl.ds(s * 624, 624)])

        @pl.when(s == 15)
        def _():
            pltpu.sync_copy(acc.at[pl.ds(9360, 640)],
                            out.at[c, pl.ds(9360, 640)])

    return seg_sum


def _make_deg():
    mesh = plsc.VectorSubcoreMesh(core_axis_name="c", subcore_axis_name="s")

    @functools.partial(
        pl.kernel,
        out_type=jax.ShapeDtypeStruct((NC, N, 16), jnp.float32),
        mesh=mesh,
        compiler_params=pltpu.CompilerParams(use_tc_tiling_on_sc=False),
        scratch_types=[
            pltpu.VMEM_SHARED((N_ACC, 16), jnp.float32),
            pltpu.VMEM((8, 128), jnp.int32),
            pltpu.VMEM((128, 16), jnp.float32),
            pltpu.SemaphoreType.DMA,
        ],
    )
    def deg_kernel(dst2d, ones_hbm, zeros16, out, accd, idx_d, ones_v, asem):
        c = lax.axis_index("c")
        s = lax.axis_index("s")
        zr = N_ACC // NS
        pltpu.sync_copy(zeros16.at[pl.ds(s * zr, zr)],
                        accd.at[pl.ds(s * zr, zr)])
        pltpu.sync_copy(ones_hbm, ones_v)
        plsc.subcore_barrier()
        tile_row0 = (c * NS + s) * ROWS_PER_TILE

        def body(g, carry):
            pltpu.sync_copy(dst2d.at[pl.ds(tile_row0 + g * 8, 8)], idx_d)
            descs = [
                pltpu.async_copy(ones_v, accd.at[idx_d.at[j]], asem,
                                 add=True)
                for j in range(8)
            ]
            for dsc in descs:
                dsc.wait()
            return carry

        lax.fori_loop(0, ROWS_PER_TILE // 8, body, 0)
        plsc.subcore_barrier()

        @pl.when(s < 15)
        def _():
            pltpu.sync_copy(accd.at[pl.ds(s * 624, 624)],
                            out.at[c, pl.ds(s * 624, 624)])

        @pl.when(s == 15)
        def _():
            pltpu.sync_copy(accd.at[pl.ds(9360, 640)],
                            out.at[c, pl.ds(9360, 640)])

    return deg_kernel


_seg64 = _make_seg_sum(64)
_seg32 = _make_seg_sum(32)
_deg = _make_deg()

_TC_R = 2000


def _tc_first(x, ws, wn, b):
    def body(x_ref, ws_ref, wn_ref, b_ref, self_ref, xna_ref, xnb_ref):
        xb = x_ref[...]
        self_ref[...] = jnp.dot(
            xb, ws_ref[...], preferred_element_type=jnp.float32) + b_ref[...]
        y = jnp.dot(xb, wn_ref[...], preferred_element_type=jnp.float32)
        w = y.shape[1] // 2
        xna_ref[...] = y[:, :w]
        xnb_ref[...] = y[:, w:]

    dk, do = ws.shape
    w = do // 2
    return pl.pallas_call(
        body,
        grid=(N // _TC_R,),
        in_specs=[
            pl.BlockSpec((_TC_R, dk), lambda i: (i, 0)),
            pl.BlockSpec((dk, do), lambda i: (0, 0)),
            pl.BlockSpec((dk, do), lambda i: (0, 0)),
            pl.BlockSpec((1, do), lambda i: (0, 0)),
        ],
        out_specs=[
            pl.BlockSpec((_TC_R, do), lambda i: (i, 0)),
            pl.BlockSpec((_TC_R, w), lambda i: (i, 0)),
            pl.BlockSpec((_TC_R, w), lambda i: (i, 0)),
        ],
        out_shape=[
            jax.ShapeDtypeStruct((N, do), jnp.float32),
            jax.ShapeDtypeStruct((N, w), jnp.float32),
            jax.ShapeDtypeStruct((N, w), jnp.float32),
        ],
    )(x, ws, wn, b)


def _tc_mid(selfp, sums, degs, ws, wn, b):
    def body(self_ref, sums_ref, degs_ref, ws_ref, wn_ref, b_ref,
             sout_ref, xna_ref, xnb_ref):
        deg = jnp.maximum(degs_ref[0, :, 0:1] + degs_ref[1, :, 0:1], 1.0)
        agg = jnp.concatenate([sums_ref[0], sums_ref[1]], axis=-1) / deg
        h = jnp.maximum(self_ref[...] + agg, 0.0)
        sout_ref[...] = jnp.dot(
            h, ws_ref[...], preferred_element_type=jnp.float32) + b_ref[...]
        y = jnp.dot(h, wn_ref[...], preferred_element_type=jnp.float32)
        w = y.shape[1] // 2
        xna_ref[...] = y[:, :w]
        xnb_ref[...] = y[:, w:]

    dk, do = ws.shape
    w = do // 2
    hw = selfp.shape[1] // 2
    return pl.pallas_call(
        body,
        grid=(N // _TC_R,),
        in_specs=[
            pl.BlockSpec((_TC_R, dk), lambda i: (i, 0)),
            pl.BlockSpec((NC, _TC_R, hw), lambda i: (0, i, 0)),
            pl.BlockSpec((NC, _TC_R, 16), lambda i: (0, i, 0)),
            pl.BlockSpec((dk, do), lambda i: (0, 0)),
            pl.BlockSpec((dk, do), lambda i: (0, 0)),
            pl.BlockSpec((1, do), lambda i: (0, 0)),
        ],
        out_specs=[
            pl.BlockSpec((_TC_R, do), lambda i: (i, 0)),
            pl.BlockSpec((_TC_R, w), lambda i: (i, 0)),
            pl.BlockSpec((_TC_R, w), lambda i: (i, 0)),
        ],
        out_shape=[
            jax.ShapeDtypeStruct((N, do), jnp.float32),
            jax.ShapeDtypeStruct((N, w), jnp.float32),
            jax.ShapeDtypeStruct((N, w), jnp.float32),
        ],
    )(selfp, sums, degs, ws, wn, b)


def _tc_final(selfp, sums, degs):
    def body(self_ref, sums_ref, degs_ref, out_ref):
        deg = jnp.maximum(degs_ref[0, :, 0:1] + degs_ref[1, :, 0:1], 1.0)
        out_ref[...] = self_ref[...] + jnp.concatenate(
            [sums_ref[0], sums_ref[1]], axis=-1) / deg

    do = selfp.shape[1]
    hw = do // 2
    return pl.pallas_call(
        body,
        grid=(N // _TC_R,),
        in_specs=[
            pl.BlockSpec((_TC_R, do), lambda i: (i, 0)),
            pl.BlockSpec((NC, _TC_R, hw), lambda i: (0, i, 0)),
            pl.BlockSpec((NC, _TC_R, 16), lambda i: (0, i, 0)),
        ],
        out_specs=pl.BlockSpec((_TC_R, do), lambda i: (i, 0)),
        out_shape=jax.ShapeDtypeStruct((N, do), jnp.float32),
    )(selfp, sums, degs)


def kernel(in_feat, edge_index, W_self0, W_neigh0, b0, W_self1, W_neigh1,
           b1, W_self2, W_neigh2, b2):
    pad = E_PAD - E
    src_p = jnp.concatenate([edge_index[0], jnp.zeros((pad,), jnp.int32)])
    dst_p = jnp.concatenate([edge_index[1], jnp.full((pad,), N, jnp.int32)])
    dst2d = dst_p.reshape(EROWS, 128)
    e2 = jnp.stack([
        src_p.reshape(-1, SEG_CHUNK),
        dst_p.reshape(-1, SEG_CHUNK),
    ], axis=1)
    zeros64 = jnp.zeros((N_ACC, 64), jnp.float32)
    zeros32 = jnp.zeros((N_ACC, 32), jnp.float32)
    zeros16 = jnp.zeros((N_ACC, 16), jnp.float32)
    ones16 = jnp.ones((128, 16), jnp.float32)

    degs = _deg(dst2d, ones16, zeros16)

    self0, xa0, xb0 = _tc_first(in_feat, W_self0, W_neigh0, b0.reshape(1, -1))
    sums0 = _seg64(xa0, xb0, e2, zeros64)
    self1, xa1, xb1 = _tc_mid(self0, sums0, degs, W_self1, W_neigh1,
                              b1.reshape(1, -1))
    sums1 = _seg64(xa1, xb1, e2, zeros64)
    self2, xa2, xb2 = _tc_mid(self1, sums1, degs, W_self2, W_neigh2,
                              b2.reshape(1, -1))
    sums2 = _seg32(xa2, xb2, e2, zeros32)
    return _tc_final(self2, sums2, degs)


# speedup vs baseline: 1.4915x; 1.0188x over previous
import functools

import jax
import jax.numpy as jnp
from jax import lax
from jax.experimental import pallas as pl
from jax.experimental.pallas import tpu as pltpu
from jax.experimental.pallas import tpu_sc as plsc

N = 10000
E = 320000
D_IN = 128
D_H = 128
D_OUT = 64

NC = 2
NS = 16

N_ACC = 10240
E_PAD = 327680
EROWS = E_PAD // 128
ROWS_PER_TILE = EROWS // (NC * NS)


SEG_CHUNK = 512
STEPS = E_PAD // (NS * SEG_CHUNK)


def _make_seg_sum(w):
    mesh = plsc.VectorSubcoreMesh(core_axis_name="c", subcore_axis_name="s")

    @functools.partial(
        pl.kernel,
        out_type=jax.ShapeDtypeStruct((NC, N, w), jnp.float32),
        mesh=mesh,
        compiler_params=pltpu.CompilerParams(use_tc_tiling_on_sc=False),
        scratch_types=[
            pltpu.VMEM_SHARED((N_ACC, w), jnp.float32),
            pltpu.VMEM((2, SEG_CHUNK), jnp.int32),
            pltpu.VMEM((2, SEG_CHUNK), jnp.int32),
            pltpu.VMEM((2, SEG_CHUNK), jnp.int32),
            pltpu.VMEM((2, SEG_CHUNK), jnp.int32),
            pltpu.VMEM((SEG_CHUNK, w), jnp.float32),
            pltpu.VMEM((SEG_CHUNK, w), jnp.float32),
            pltpu.SemaphoreType.DMA,
            pltpu.SemaphoreType.DMA,
            pltpu.SemaphoreType.DMA,
        ],
    )
    def seg_sum(tablea, tableb, e2, zeros, gate, out, acc, i0, i1, i2, i3,
                r0, r1, isem, gsem, asem):
        del gate
        c = lax.axis_index("c")
        s = lax.axis_index("s")
        idxb = [i0, i1, i2, i3]
        rows = [r0, r1]
        zr = N_ACC // NS
        pltpu.sync_copy(zeros.at[pl.ds(s * zr, zr)], acc.at[pl.ds(s * zr, zr)])
        plsc.subcore_barrier()
        t0 = s * STEPS

        def fire_idx(e, et):
            pltpu.async_copy(e2.at[t0 + et], idxb[e % 4], isem)

        def wait_idx(e):
            pltpu.make_async_copy(e2.at[t0], idxb[e % 4], isem).wait()

        def fire_gather(e):
            @pl.when(c == 0)
            def _():
                pltpu.async_copy(tablea.at[idxb[e % 4].at[0]], rows[e % 2],
                                 gsem)

            @pl.when(c == 1)
            def _():
                pltpu.async_copy(tableb.at[idxb[e % 4].at[0]], rows[e % 2],
                                 gsem)

        def wait_gather(e):
            pltpu.make_async_copy(tablea.at[idxb[e % 4].at[0]], rows[e % 2],
                                  gsem).wait()

        def fire_add(e):
            pltpu.async_copy(rows[e % 2], acc.at[idxb[e % 4].at[1]], asem,
                             add=True)

        def wait_add(e):
            pltpu.make_async_copy(rows[e % 2], acc.at[idxb[e % 4].at[1]],
                                  asem).wait()

        def st(e, et, *, wait_prev=True, next_gather=True, next2_idx=True):
            wait_gather(e)
            fire_add(e)
            if wait_prev:
                wait_add(e - 1)
            if next_gather:
                wait_idx(e + 1)
                fire_gather(e + 1)
            if next2_idx:
                fire_idx(e + 2, et + 2)

        pltpu.sync_copy(e2.at[t0], i0)
        fire_gather(0)
        fire_idx(1, 1)
        st(0, 0, wait_prev=False)
        st(1, 1)
        st(2, 2)
        st(3, 3)

        def body(g, carry):
            et = 4 * g
            st(0, et)
            st(1, et + 1)
            st(2, et + 2)
            st(3, et + 3)
            return carry

        lax.fori_loop(1, STEPS // 4 - 1, body, 0)

        st(0, STEPS - 4)
        st(1, STEPS - 3)
        st(2, STEPS - 2, next2_idx=False)
        st(3, STEPS - 1, next_gather=False, next2_idx=False)
        wait_add(3)

        plsc.subcore_barrier()
        @pl.when(s < 15)
        def _():
            pltpu.sync_copy(acc.at[pl.ds(s * 624, 624)],
                            out.at[c, pl.ds(s * 624, 624)])

        @pl.when(s == 15)
        def _():
            pltpu.sync_copy(acc.at[pl.ds(9360, 640)],
                            out.at[c, pl.ds(9360, 640)])

    return seg_sum


def _make_deg():
    mesh = plsc.VectorSubcoreMesh(core_axis_name="c", subcore_axis_name="s")

    @functools.partial(
        pl.kernel,
        out_type=jax.ShapeDtypeStruct((NC, N, 16), jnp.float32),
        mesh=mesh,
        compiler_params=pltpu.CompilerParams(use_tc_tiling_on_sc=False),
        scratch_types=[
            pltpu.VMEM_SHARED((N_ACC, 16), jnp.float32),
            pltpu.VMEM((8, 128), jnp.int32),
            pltpu.VMEM((128, 16), jnp.float32),
            pltpu.SemaphoreType.DMA,
        ],
    )
    def deg_kernel(dst2d, ones_hbm, zeros16, out, accd, idx_d, ones_v, asem):
        c = lax.axis_index("c")
        s = lax.axis_index("s")
        zr = N_ACC // NS
        pltpu.sync_copy(zeros16.at[pl.ds(s * zr, zr)],
                        accd.at[pl.ds(s * zr, zr)])
        pltpu.sync_copy(ones_hbm, ones_v)
        plsc.subcore_barrier()
        tile_row0 = (c * NS + s) * ROWS_PER_TILE

        def body(g, carry):
            pltpu.sync_copy(dst2d.at[pl.ds(tile_row0 + g * 8, 8)], idx_d)
            descs = [
                pltpu.async_copy(ones_v, accd.at[idx_d.at[j]], asem,
                                 add=True)
                for j in range(8)
            ]
            for dsc in descs:
                dsc.wait()
            return carry

        lax.fori_loop(0, ROWS_PER_TILE // 8, body, 0)
        plsc.subcore_barrier()

        @pl.when(s < 15)
        def _():
            pltpu.sync_copy(accd.at[pl.ds(s * 624, 624)],
                            out.at[c, pl.ds(s * 624, 624)])

        @pl.when(s == 15)
        def _():
            pltpu.sync_copy(accd.at[pl.ds(9360, 640)],
                            out.at[c, pl.ds(9360, 640)])

    return deg_kernel


_seg64 = _make_seg_sum(64)
_seg32 = _make_seg_sum(32)
_deg = _make_deg()

_TC_R = 2000


def _tc_first(x, ws, wn, b):
    def body(x_ref, ws_ref, wn_ref, b_ref, self_ref, xna_ref, xnb_ref):
        xb = x_ref[...]
        self_ref[...] = jnp.dot(
            xb, ws_ref[...], preferred_element_type=jnp.float32) + b_ref[...]
        y = jnp.dot(xb, wn_ref[...], preferred_element_type=jnp.float32)
        w = y.shape[1] // 2
        xna_ref[...] = y[:, :w]
        xnb_ref[...] = y[:, w:]

    dk, do = ws.shape
    w = do // 2
    return pl.pallas_call(
        body,
        grid=(N // _TC_R,),
        in_specs=[
            pl.BlockSpec((_TC_R, dk), lambda i: (i, 0)),
            pl.BlockSpec((dk, do), lambda i: (0, 0)),
            pl.BlockSpec((dk, do), lambda i: (0, 0)),
            pl.BlockSpec((1, do), lambda i: (0, 0)),
        ],
        out_specs=[
            pl.BlockSpec((_TC_R, do), lambda i: (i, 0)),
            pl.BlockSpec((_TC_R, w), lambda i: (i, 0)),
            pl.BlockSpec((_TC_R, w), lambda i: (i, 0)),
        ],
        out_shape=[
            jax.ShapeDtypeStruct((N, do), jnp.float32),
            jax.ShapeDtypeStruct((N, w), jnp.float32),
            jax.ShapeDtypeStruct((N, w), jnp.float32),
        ],
    )(x, ws, wn, b)


def _tc_mid(selfp, sums, degs, ws, wn, b):
    def body(self_ref, sums_ref, degs_ref, ws_ref, wn_ref, b_ref,
             sout_ref, xna_ref, xnb_ref):
        deg = jnp.maximum(degs_ref[0, :, 0:1] + degs_ref[1, :, 0:1], 1.0)
        agg = jnp.concatenate([sums_ref[0], sums_ref[1]], axis=-1) / deg
        h = jnp.maximum(self_ref[...] + agg, 0.0)
        sout_ref[...] = jnp.dot(
            h, ws_ref[...], preferred_element_type=jnp.float32) + b_ref[...]
        y = jnp.dot(h, wn_ref[...], preferred_element_type=jnp.float32)
        w = y.shape[1] // 2
        xna_ref[...] = y[:, :w]
        xnb_ref[...] = y[:, w:]

    dk, do = ws.shape
    w = do // 2
    hw = selfp.shape[1] // 2
    return pl.pallas_call(
        body,
        grid=(N // _TC_R,),
        in_specs=[
            pl.BlockSpec((_TC_R, dk), lambda i: (i, 0)),
            pl.BlockSpec((NC, _TC_R, hw), lambda i: (0, i, 0)),
            pl.BlockSpec((NC, _TC_R, 16), lambda i: (0, i, 0)),
            pl.BlockSpec((dk, do), lambda i: (0, 0)),
            pl.BlockSpec((dk, do), lambda i: (0, 0)),
            pl.BlockSpec((1, do), lambda i: (0, 0)),
        ],
        out_specs=[
            pl.BlockSpec((_TC_R, do), lambda i: (i, 0)),
            pl.BlockSpec((_TC_R, w), lambda i: (i, 0)),
            pl.BlockSpec((_TC_R, w), lambda i: (i, 0)),
        ],
        out_shape=[
            jax.ShapeDtypeStruct((N, do), jnp.float32),
            jax.ShapeDtypeStruct((N, w), jnp.float32),
            jax.ShapeDtypeStruct((N, w), jnp.float32),
        ],
    )(selfp, sums, degs, ws, wn, b)


def _tc_final(selfp, sums, degs):
    def body(self_ref, sums_ref, degs_ref, out_ref):
        deg = jnp.maximum(degs_ref[0, :, 0:1] + degs_ref[1, :, 0:1], 1.0)
        out_ref[...] = self_ref[...] + jnp.concatenate(
            [sums_ref[0], sums_ref[1]], axis=-1) / deg

    do = selfp.shape[1]
    hw = do // 2
    return pl.pallas_call(
        body,
        grid=(N // _TC_R,),
        in_specs=[
            pl.BlockSpec((_TC_R, do), lambda i: (i, 0)),
            pl.BlockSpec((NC, _TC_R, hw), lambda i: (0, i, 0)),
            pl.BlockSpec((NC, _TC_R, 16), lambda i: (0, i, 0)),
        ],
        out_specs=pl.BlockSpec((_TC_R, do), lambda i: (i, 0)),
        out_shape=jax.ShapeDtypeStruct((N, do), jnp.float32),
    )(selfp, sums, degs)


def kernel(in_feat, edge_index, W_self0, W_neigh0, b0, W_self1, W_neigh1,
           b1, W_self2, W_neigh2, b2):
    pad = E_PAD - E
    src_p = jnp.concatenate([edge_index[0], jnp.zeros((pad,), jnp.int32)])
    dst_p = jnp.concatenate([edge_index[1], jnp.full((pad,), N, jnp.int32)])
    dst2d = dst_p.reshape(EROWS, 128)
    e2 = jnp.stack([
        src_p.reshape(-1, SEG_CHUNK),
        dst_p.reshape(-1, SEG_CHUNK),
    ], axis=1)
    zeros64 = jnp.zeros((N_ACC, 64), jnp.float32)
    zeros32 = jnp.zeros((N_ACC, 32), jnp.float32)
    zeros16 = jnp.zeros((N_ACC, 16), jnp.float32)
    ones16 = jnp.ones((128, 16), jnp.float32)

    degs = _deg(dst2d, ones16, zeros16)

    self0, xa0, xb0 = _tc_first(in_feat, W_self0, W_neigh0, b0.reshape(1, -1))
    sums0 = _seg64(xa0, xb0, e2, zeros64, degs)
    self1, xa1, xb1 = _tc_mid(self0, sums0, degs, W_self1, W_neigh1,
                              b1.reshape(1, -1))
    sums1 = _seg64(xa1, xb1, e2, zeros64, degs)
    self2, xa2, xb2 = _tc_mid(self1, sums1, degs, W_self2, W_neigh2,
                              b2.reshape(1, -1))
    sums2 = _seg32(xa2, xb2, e2, zeros32, degs)
    return _tc_final(self2, sums2, degs)


# speedup vs baseline: 1.5396x; 1.0322x over previous
import functools

import jax
import jax.numpy as jnp
from jax import lax
from jax.experimental import pallas as pl
from jax.experimental.pallas import tpu as pltpu
from jax.experimental.pallas import tpu_sc as plsc

N = 10000
E = 320000
D_IN = 128
D_H = 128
D_OUT = 64

NC = 2
NS = 16

N_ACC = 10240
E_PAD = 327680
EROWS = E_PAD // 128
ROWS_PER_TILE = EROWS // (NC * NS)


SEG_CHUNK = 256
STEPS = E_PAD // (NS * SEG_CHUNK)
NRB = 4
NIB = 8
UNROLL = 8


def _make_seg_sum(w):
    mesh = plsc.VectorSubcoreMesh(core_axis_name="c", subcore_axis_name="s")

    @functools.partial(
        pl.kernel,
        out_type=jax.ShapeDtypeStruct((NC, N, w), jnp.float32),
        mesh=mesh,
        compiler_params=pltpu.CompilerParams(use_tc_tiling_on_sc=False),
        scratch_types=(
            [pltpu.VMEM_SHARED((N_ACC, w), jnp.float32)]
            + [pltpu.VMEM((2, SEG_CHUNK), jnp.int32) for _ in range(NIB)]
            + [pltpu.VMEM((SEG_CHUNK, w), jnp.float32) for _ in range(NRB)]
            + [pltpu.SemaphoreType.DMA] * 3
        ),
    )
    def seg_sum(tablea, tableb, e2, zeros, gate, out, acc, *rest):
        del gate
        idxb = list(rest[:NIB])
        rows = list(rest[NIB:NIB + NRB])
        isem, gsem, asem = rest[NIB + NRB:]
        c = lax.axis_index("c")
        s = lax.axis_index("s")
        zr = N_ACC // NS
        pltpu.sync_copy(zeros.at[pl.ds(s * zr, zr)], acc.at[pl.ds(s * zr, zr)])
        plsc.subcore_barrier()
        t0 = s * STEPS

        def fire_idx(e, et):
            pltpu.async_copy(e2.at[t0 + et], idxb[e % NIB], isem)

        def wait_idx(e):
            pltpu.make_async_copy(e2.at[t0], idxb[e % NIB], isem).wait()

        def fire_gather(e):
            @pl.when(c == 0)
            def _():
                pltpu.async_copy(tablea.at[idxb[e % NIB].at[0]],
                                 rows[e % NRB], gsem)

            @pl.when(c == 1)
            def _():
                pltpu.async_copy(tableb.at[idxb[e % NIB].at[0]],
                                 rows[e % NRB], gsem)

        def wait_gather(e):
            pltpu.make_async_copy(tablea.at[idxb[e % NIB].at[0]],
                                  rows[e % NRB], gsem).wait()

        def fire_add(e):
            pltpu.async_copy(rows[e % NRB], acc.at[idxb[e % NIB].at[1]], asem,
                             add=True)

        def wait_add(e):
            pltpu.make_async_copy(rows[e % NRB], acc.at[idxb[e % NIB].at[1]],
                                  asem).wait()

        def st(e, et):
            wait_gather(e)
            fire_add(e)
            if e >= 1:
                wait_add(e - 1)
            if e + NRB - 1 <= STEPS - 1:
                wait_idx(e + NRB - 1)
                fire_gather(e + NRB - 1)
            if e + NRB <= STEPS - 1:
                fire_idx(e + NRB, et + NRB)

        for k in range(NRB):
            fire_idx(k, k)
        for k in range(NRB - 1):
            wait_idx(k)
            fire_gather(k)
        for k in range(UNROLL):
            st(k, k)

        def body(g, carry):
            et = UNROLL * g
            for sub in range(UNROLL):
                st(UNROLL + sub, et + sub)
            return carry

        lax.fori_loop(1, STEPS // UNROLL - 1, body, 0)

        for k in range(UNROLL):
            st(STEPS - UNROLL + k, STEPS - UNROLL + k)
        wait_add(STEPS - 1)

        plsc.subcore_barrier()
        @pl.when(s < 15)
        def _():
            pltpu.sync_copy(acc.at[pl.ds(s * 624, 624)],
                            out.at[c, pl.ds(s * 624, 624)])

        @pl.when(s == 15)
        def _():
            pltpu.sync_copy(acc.at[pl.ds(9360, 640)],
                            out.at[c, pl.ds(9360, 640)])

    return seg_sum


def _make_deg():
    mesh = plsc.VectorSubcoreMesh(core_axis_name="c", subcore_axis_name="s")

    @functools.partial(
        pl.kernel,
        out_type=jax.ShapeDtypeStruct((NC, N, 16), jnp.float32),
        mesh=mesh,
        compiler_params=pltpu.CompilerParams(use_tc_tiling_on_sc=False),
        scratch_types=[
            pltpu.VMEM_SHARED((N_ACC, 16), jnp.float32),
            pltpu.VMEM((8, 128), jnp.int32),
            pltpu.VMEM((128, 16), jnp.float32),
            pltpu.SemaphoreType.DMA,
        ],
    )
    def deg_kernel(dst2d, ones_hbm, zeros16, out, accd, idx_d, ones_v, asem):
        c = lax.axis_index("c")
        s = lax.axis_index("s")
        zr = N_ACC // NS
        pltpu.sync_copy(zeros16.at[pl.ds(s * zr, zr)],
                        accd.at[pl.ds(s * zr, zr)])
        pltpu.sync_copy(ones_hbm, ones_v)
        plsc.subcore_barrier()
        tile_row0 = (c * NS + s) * ROWS_PER_TILE

        def body(g, carry):
            pltpu.sync_copy(dst2d.at[pl.ds(tile_row0 + g * 8, 8)], idx_d)
            descs = [
                pltpu.async_copy(ones_v, accd.at[idx_d.at[j]], asem,
                                 add=True)
                for j in range(8)
            ]
            for dsc in descs:
                dsc.wait()
            return carry

        lax.fori_loop(0, ROWS_PER_TILE // 8, body, 0)
        plsc.subcore_barrier()

        @pl.when(s < 15)
        def _():
            pltpu.sync_copy(accd.at[pl.ds(s * 624, 624)],
                            out.at[c, pl.ds(s * 624, 624)])

        @pl.when(s == 15)
        def _():
            pltpu.sync_copy(accd.at[pl.ds(9360, 640)],
                            out.at[c, pl.ds(9360, 640)])

    return deg_kernel


_seg64 = _make_seg_sum(64)
_seg32 = _make_seg_sum(32)
_deg = _make_deg()

_TC_R = 2000


def _tc_first(x, ws, wn, b):
    def body(x_ref, ws_ref, wn_ref, b_ref, self_ref, xna_ref, xnb_ref):
        xb = x_ref[...]
        self_ref[...] = jnp.dot(
            xb, ws_ref[...], preferred_element_type=jnp.float32) + b_ref[...]
        y = jnp.dot(xb, wn_ref[...], preferred_element_type=jnp.float32)
        w = y.shape[1] // 2
        xna_ref[...] = y[:, :w]
        xnb_ref[...] = y[:, w:]

    dk, do = ws.shape
    w = do // 2
    return pl.pallas_call(
        body,
        grid=(N // _TC_R,),
        in_specs=[
            pl.BlockSpec((_TC_R, dk), lambda i: (i, 0)),
            pl.BlockSpec((dk, do), lambda i: (0, 0)),
            pl.BlockSpec((dk, do), lambda i: (0, 0)),
            pl.BlockSpec((1, do), lambda i: (0, 0)),
        ],
        out_specs=[
            pl.BlockSpec((_TC_R, do), lambda i: (i, 0)),
            pl.BlockSpec((_TC_R, w), lambda i: (i, 0)),
            pl.BlockSpec((_TC_R, w), lambda i: (i, 0)),
        ],
        out_shape=[
            jax.ShapeDtypeStruct((N, do), jnp.float32),
            jax.ShapeDtypeStruct((N, w), jnp.float32),
            jax.ShapeDtypeStruct((N, w), jnp.float32),
        ],
    )(x, ws, wn, b)


def _tc_mid(selfp, sums, degs, ws, wn, b):
    def body(self_ref, sums_ref, degs_ref, ws_ref, wn_ref, b_ref,
             sout_ref, xna_ref, xnb_ref):
        deg = jnp.maximum(degs_ref[0, :, 0:1] + degs_ref[1, :, 0:1], 1.0)
        agg = jnp.concatenate([sums_ref[0], sums_ref[1]], axis=-1) / deg
        h = jnp.maximum(self_ref[...] + agg, 0.0)
        sout_ref[...] = jnp.dot(
            h, ws_ref[...], preferred_element_type=jnp.float32) + b_ref[...]
        y = jnp.dot(h, wn_ref[...], preferred_element_type=jnp.float32)
        w = y.shape[1] // 2
        xna_ref[...] = y[:, :w]
        xnb_ref[...] = y[:, w:]

    dk, do = ws.shape
    w = do // 2
    hw = selfp.shape[1] // 2
    return pl.pallas_call(
        body,
        grid=(N // _TC_R,),
        in_specs=[
            pl.BlockSpec((_TC_R, dk), lambda i: (i, 0)),
            pl.BlockSpec((NC, _TC_R, hw), lambda i: (0, i, 0)),
            pl.BlockSpec((NC, _TC_R, 16), lambda i: (0, i, 0)),
            pl.BlockSpec((dk, do), lambda i: (0, 0)),
            pl.BlockSpec((dk, do), lambda i: (0, 0)),
            pl.BlockSpec((1, do), lambda i: (0, 0)),
        ],
        out_specs=[
            pl.BlockSpec((_TC_R, do), lambda i: (i, 0)),
            pl.BlockSpec((_TC_R, w), lambda i: (i, 0)),
            pl.BlockSpec((_TC_R, w), lambda i: (i, 0)),
        ],
        out_shape=[
            jax.ShapeDtypeStruct((N, do), jnp.float32),
            jax.ShapeDtypeStruct((N, w), jnp.float32),
            jax.ShapeDtypeStruct((N, w), jnp.float32),
        ],
    )(selfp, sums, degs, ws, wn, b)


def _tc_final(selfp, sums, degs):
    def body(self_ref, sums_ref, degs_ref, out_ref):
        deg = jnp.maximum(degs_ref[0, :, 0:1] + degs_ref[1, :, 0:1], 1.0)
        out_ref[...] = self_ref[...] + jnp.concatenate(
            [sums_ref[0], sums_ref[1]], axis=-1) / deg

    do = selfp.shape[1]
    hw = do // 2
    return pl.pallas_call(
        body,
        grid=(N // _TC_R,),
        in_specs=[
            pl.BlockSpec((_TC_R, do), lambda i: (i, 0)),
            pl.BlockSpec((NC, _TC_R, hw), lambda i: (0, i, 0)),
            pl.BlockSpec((NC, _TC_R, 16), lambda i: (0, i, 0)),
        ],
        out_specs=pl.BlockSpec((_TC_R, do), lambda i: (i, 0)),
        out_shape=jax.ShapeDtypeStruct((N, do), jnp.float32),
    )(selfp, sums, degs)


def kernel(in_feat, edge_index, W_self0, W_neigh0, b0, W_self1, W_neigh1,
           b1, W_self2, W_neigh2, b2):
    pad = E_PAD - E
    src_p = jnp.concatenate([edge_index[0], jnp.zeros((pad,), jnp.int32)])
    dst_p = jnp.concatenate([edge_index[1], jnp.full((pad,), N, jnp.int32)])
    dst2d = dst_p.reshape(EROWS, 128)
    e2 = jnp.stack([
        src_p.reshape(-1, SEG_CHUNK),
        dst_p.reshape(-1, SEG_CHUNK),
    ], axis=1)
    zeros64 = jnp.zeros((N_ACC, 64), jnp.float32)
    zeros32 = jnp.zeros((N_ACC, 32), jnp.float32)
    zeros16 = jnp.zeros((N_ACC, 16), jnp.float32)
    ones16 = jnp.ones((128, 16), jnp.float32)

    degs = _deg(dst2d, ones16, zeros16)

    self0, xa0, xb0 = _tc_first(in_feat, W_self0, W_neigh0, b0.reshape(1, -1))
    sums0 = _seg64(xa0, xb0, e2, zeros64, degs)
    self1, xa1, xb1 = _tc_mid(self0, sums0, degs, W_self1, W_neigh1,
                              b1.reshape(1, -1))
    sums1 = _seg64(xa1, xb1, e2, zeros64, degs)
    self2, xa2, xb2 = _tc_mid(self1, sums1, degs, W_self2, W_neigh2,
                              b2.reshape(1, -1))
    sums2 = _seg32(xa2, xb2, e2, zeros32, degs)
    return _tc_final(self2, sums2, degs)
